# baseline (speedup 1.0000x reference)
import functools
import math

import jax
import jax.numpy as jnp
from jax import lax
from jax.experimental import pallas as pl
from jax.experimental.pallas import tpu as pltpu
from jax.experimental.pallas import tpu_sc as plsc

N = 10000
E = 160000
NFEAT = 128
HID = 64
NCLASS = 40
NLAYER = 8
ALPHA = 0.1
LAM = 0.5
TAU = 0.5
LAMDA = 1.0

NC = 2
NS = 16
NW = NC * NS
CHUNK = 128
NBUF = 2
CNT0 = 20
CNT1 = 60
CNTMAX = max(CNT0, CNT1)
TOTCH = NS * (CNT0 + CNT1)
EPAD = TOTCH * CHUNK
ACCN = 10240
ROWS_PER_W = ACCN // NS

RB = 2000
NPAD = 10240
SIMB = 256


def _spmm_body(h_hbm, src_hbm, dst_hbm, wts_hbm, out_hbm,
               src_v, dst_v, wts_v, rg_v, rs_v, acc,
               gsem, ssem):
    c = lax.axis_index("c")
    s = lax.axis_index("s")

    zeros16 = jnp.zeros((16,), jnp.float32)

    def zrow(i, carry):
        for k in range(HID // 16):
            rs_v[0, i, pl.ds(k * 16, 16)] = zeros16
        return carry

    lax.fori_loop(0, CHUNK, zrow, 0)

    base = s * ROWS_PER_W

    def zacc(i, carry):
        pltpu.sync_copy(rs_v.at[0], acc.at[pl.ds(base + i * CHUNK, CHUNK)])
        return carry

    lax.fori_loop(0, ROWS_PER_W // CHUNK, zacc, 0)

    def run_edges(start, cnt):
        pltpu.sync_copy(src_hbm.at[pl.ds(start, cnt)], src_v.at[pl.ds(0, cnt)])
        pltpu.sync_copy(dst_hbm.at[pl.ds(start, cnt)], dst_v.at[pl.ds(0, cnt)])
        pltpu.sync_copy(wts_hbm.at[pl.ds(start, cnt)], wts_v.at[pl.ds(0, cnt)])
        plsc.subcore_barrier()

        def start_gather(j, b):
            pltpu.async_copy(h_hbm.at[src_v.at[j]], rg_v.at[b], gsem.at[b])

        def wait_gather(j, b):
            pltpu.make_async_copy(h_hbm.at[src_v.at[j]], rg_v.at[b],
                                  gsem.at[b]).wait()

        def wait_scatter(j, b):
            pltpu.make_async_copy(rs_v.at[b], acc.at[dst_v.at[j]],
                                  ssem.at[b]).wait()

        for b in range(NBUF):
            start_gather(b, b)

        def wave(p, carry):
            for b in range(NBUF):
                j = NBUF * p + b
                wait_gather(j, b)

                @pl.when(j >= NBUF)
                def _():
                    wait_scatter(j - NBUF, b)

                zlane = lax.iota(jnp.int32, 16) * 0

                @plsc.parallel_loop(0, CHUNK // 16, unroll=2)
                def _scale_grp(g):
                    w16 = wts_v[j, pl.ds(g * 16, 16)]
                    for l in range(16):
                        wt = w16.at[zlane + l].get(mode="promise_in_bounds")
                        e = g * 16 + l
                        for k in range(HID // 16):
                            rs_v[b, e, pl.ds(k * 16, 16)] = \
                                rg_v[b, e, pl.ds(k * 16, 16)] * wt

                @pl.when(j + NBUF < cnt)
                def _():
                    start_gather(j + NBUF, b)

                pltpu.async_copy(rs_v.at[b], acc.at[dst_v.at[j]], ssem.at[b],
                                 add=True)
            return carry

        lax.fori_loop(0, cnt // NBUF, wave, 0)
        for b in range(NBUF):
            wait_scatter(cnt - NBUF + b, b)

    @pl.when(c == 0)
    def _():
        run_edges(s * CNT0, CNT0)

    @pl.when(c == 1)
    def _():
        run_edges(NS * CNT0 + s * CNT1, CNT1)

    plsc.subcore_barrier()

    def wout(i, carry):
        pltpu.sync_copy(acc.at[pl.ds(base + i * CHUNK, CHUNK)],
                        out_hbm.at[c, pl.ds(base + i * CHUNK, CHUNK)])
        return carry

    lax.fori_loop(0, ROWS_PER_W // CHUNK, wout, 0)


@functools.lru_cache(maxsize=None)
def _spmm_call():
    return pl.kernel(
        _spmm_body,
        out_type=jax.ShapeDtypeStruct((NC, ACCN, HID), jnp.float32),
        mesh=plsc.VectorSubcoreMesh(core_axis_name="c", subcore_axis_name="s",
                                    num_cores=NC, num_subcores=NS),
        scratch_types=[
            pltpu.VMEM((CNTMAX, CHUNK), jnp.int32),
            pltpu.VMEM((CNTMAX, CHUNK), jnp.int32),
            pltpu.VMEM((CNTMAX, CHUNK), jnp.float32),
            pltpu.VMEM((NBUF, CHUNK, HID), jnp.float32),
            pltpu.VMEM((NBUF, CHUNK, HID), jnp.float32),
            pltpu.VMEM_SHARED((ACCN, HID), jnp.float32),
            pltpu.SemaphoreType.DMA((NBUF,)),
            pltpu.SemaphoreType.DMA((NBUF,)),
        ],
        compiler_params=pltpu.CompilerParams(use_tc_tiling_on_sc=False),
    )


def _h0_kern(x_ref, w_ref, b_ref, o_ref):
    acc = jnp.dot(x_ref[...], w_ref[...], preferred_element_type=jnp.float32)
    o_ref[...] = jnp.maximum(acc + b_ref[...], 0.0)


def _layer_kern(p0_ref, p1_ref, h0_ref, hl_ref, wc_ref, o_ref, *, theta, beta):
    sup = (1.0 - ALPHA) * (p0_ref[0] + p1_ref[0]) + ALPHA * h0_ref[...]
    out = theta * jnp.dot(sup, wc_ref[...], preferred_element_type=jnp.float32) \
        + (1.0 - theta) * sup
    o_ref[...] = beta * jnp.maximum(out, 0.0) + (1.0 - beta) * hl_ref[...]


def _zn_kern(z_ref, o_ref):
    z = z_ref[...]
    n2 = jnp.sum(z * z, axis=1, keepdims=True)
    n = jnp.sqrt(n2)
    o_ref[...] = z / jnp.maximum(n, 1e-12)


def _sim_kern(zb_ref, znt_ref, loss_ref, acc_ref):
    i = pl.program_id(0)
    zb = zb_ref[...]
    s = jnp.dot(zb, znt_ref[...], preferred_element_type=jnp.float32)
    es = jnp.exp(s * (1.0 / TAU))
    rowsum = jnp.sum(es, axis=1, keepdims=True) - float(NPAD - N)
    d = jnp.sum(zb * zb, axis=1, keepdims=True)
    diag = jnp.exp(d * (1.0 / TAU))
    neg = rowsum - diag
    ct = -jnp.log(diag / (2.0 * neg))
    rid = i * SIMB + lax.broadcasted_iota(jnp.int32, (SIMB, 1), 0)
    blk = jnp.sum(jnp.where(rid < N, ct, 0.0))

    @pl.when(i == 0)
    def _():
        acc_ref[0] = 0.0

    acc_ref[0] += blk

    @pl.when(i == NPAD // SIMB - 1)
    def _():
        loss_ref[0, 0] = acc_ref[0] / float(N)


def _logits_kern(h_ref, w_ref, b_ref, o_ref):
    logits = jnp.dot(h_ref[...], w_ref[...],
                     preferred_element_type=jnp.float32) + b_ref[...]
    m = jnp.max(logits, axis=1, keepdims=True)
    sh = logits - m
    lse = jnp.log(jnp.sum(jnp.exp(sh), axis=1, keepdims=True))
    o_ref[...] = sh - lse


_h0_call = pl.pallas_call(
    _h0_kern,
    grid=(N // RB,),
    in_specs=[
        pl.BlockSpec((RB, NFEAT), lambda i: (i, 0)),
        pl.BlockSpec((NFEAT, HID), lambda i: (0, 0)),
        pl.BlockSpec((1, HID), lambda i: (0, 0)),
    ],
    out_specs=pl.BlockSpec((RB, HID), lambda i: (i, 0)),
    out_shape=jax.ShapeDtypeStruct((N, HID), jnp.float32),
)


def _layer_call(theta, beta):
    return pl.pallas_call(
        functools.partial(_layer_kern, theta=theta, beta=beta),
        grid=(N // RB,),
        in_specs=[
            pl.BlockSpec((1, RB, HID), lambda i: (0, i, 0)),
            pl.BlockSpec((1, RB, HID), lambda i: (1, i, 0)),
            pl.BlockSpec((RB, HID), lambda i: (i, 0)),
            pl.BlockSpec((RB, HID), lambda i: (i, 0)),
            pl.BlockSpec((HID, HID), lambda i: (0, 0)),
        ],
        out_specs=pl.BlockSpec((RB, HID), lambda i: (i, 0)),
        out_shape=jax.ShapeDtypeStruct((N, HID), jnp.float32),
    )


_zn_call = pl.pallas_call(
    _zn_kern,
    grid=(NPAD // 2048,),
    in_specs=[pl.BlockSpec((2048, HID), lambda i: (i, 0))],
    out_specs=pl.BlockSpec((2048, HID), lambda i: (i, 0)),
    out_shape=jax.ShapeDtypeStruct((NPAD, HID), jnp.float32),
)

_sim_call = pl.pallas_call(
    _sim_kern,
    grid=(NPAD // SIMB,),
    in_specs=[
        pl.BlockSpec((SIMB, HID), lambda i: (i, 0)),
        pl.BlockSpec((HID, NPAD), lambda i: (0, 0)),
    ],
    out_specs=pl.BlockSpec(memory_space=pltpu.SMEM),
    out_shape=jax.ShapeDtypeStruct((1, 1), jnp.float32),
    scratch_shapes=[pltpu.SMEM((1,), jnp.float32)],
)

_logits_call = pl.pallas_call(
    _logits_kern,
    grid=(N // RB,),
    in_specs=[
        pl.BlockSpec((RB, HID), lambda i: (i, 0)),
        pl.BlockSpec((HID, 128), lambda i: (0, 0)),
        pl.BlockSpec((1, 128), lambda i: (0, 0)),
    ],
    out_specs=pl.BlockSpec((RB, 128), lambda i: (i, 0)),
    out_shape=jax.ShapeDtypeStruct((N, 128), jnp.float32),
)


def kernel(x, edge_index, edge_weight, Wc, W0, b0, W1, b1):
    h0 = _h0_call(x, W0, b0.reshape(1, HID))

    dst = edge_index[0]
    src = edge_index[1]
    pad = EPAD - E
    src_p = jnp.concatenate([src, jnp.zeros((pad,), jnp.int32)]).reshape(TOTCH, CHUNK)
    dst_p = jnp.concatenate([dst, jnp.zeros((pad,), jnp.int32)]).reshape(TOTCH, CHUNK)
    wts_p = jnp.concatenate([edge_weight, jnp.zeros((pad,), jnp.float32)]).reshape(TOTCH, CHUNK)

    last = h0
    for i in range(NLAYER):
        l = i + 1
        theta = math.log(LAM / l + 1.0)
        beta = math.log(LAMDA / l + 1.0)
        parts = _spmm_call()(last, src_p, dst_p, wts_p)
        last = _layer_call(theta, beta)(parts, parts, h0, last, Wc[i])

    lastp = jnp.pad(last, ((0, NPAD - N), (0, 0)))
    znp = _zn_call(lastp)
    loss = _sim_call(znp, znp.T)[0, 0]

    W1p = jnp.pad(W1, ((0, 0), (0, 128 - NCLASS)))
    b1p = jnp.pad(b1, (0, 128 - NCLASS), constant_values=-1e30).reshape(1, 128)
    logp = _logits_call(last, W1p, b1p)[:, :NCLASS]
    return (logp, loss)


# speedup vs baseline: 1.1074x; 1.1074x over previous
import functools
import math

import jax
import jax.numpy as jnp
from jax import lax
from jax.experimental import pallas as pl
from jax.experimental.pallas import tpu as pltpu
from jax.experimental.pallas import tpu_sc as plsc

N = 10000
E = 160000
NFEAT = 128
HID = 64
NCLASS = 40
NLAYER = 8
ALPHA = 0.1
LAM = 0.5
TAU = 0.5
LAMDA = 1.0

NC = 2
NS = 16
NW = NC * NS
CHUNK = 128
NBUF = 2
CNT0 = 40
CNT1 = 40
CNTMAX = max(CNT0, CNT1)
TOTCH = NS * (CNT0 + CNT1)
EPAD = TOTCH * CHUNK
ACCN = 10240
ROWS_PER_W = ACCN // NS

RB = 2000
NPAD = 10240
SIMB = 256


def _spmm_body(h_hbm, src_hbm, dst_hbm, wts_hbm, out_hbm,
               src_v, dst_v, wts_v, rg_v, rs_v, acc,
               gsem, ssem):
    c = lax.axis_index("c")
    s = lax.axis_index("s")

    zeros16 = jnp.zeros((16,), jnp.float32)

    def zrow(i, carry):
        for k in range(HID // 16):
            rs_v[0, i, pl.ds(k * 16, 16)] = zeros16
        return carry

    lax.fori_loop(0, CHUNK, zrow, 0)

    base = s * ROWS_PER_W

    def zacc(i, carry):
        pltpu.sync_copy(rs_v.at[0], acc.at[pl.ds(base + i * CHUNK, CHUNK)])
        return carry

    lax.fori_loop(0, ROWS_PER_W // CHUNK, zacc, 0)

    def run_edges(start, cnt):
        pltpu.sync_copy(src_hbm.at[pl.ds(start, cnt)], src_v.at[pl.ds(0, cnt)])
        pltpu.sync_copy(dst_hbm.at[pl.ds(start, cnt)], dst_v.at[pl.ds(0, cnt)])
        pltpu.sync_copy(wts_hbm.at[pl.ds(start, cnt)], wts_v.at[pl.ds(0, cnt)])
        plsc.subcore_barrier()

        def start_gather(j, b):
            pltpu.async_copy(h_hbm.at[src_v.at[j]], rg_v.at[b], gsem.at[b])

        def wait_gather(j, b):
            pltpu.make_async_copy(h_hbm.at[src_v.at[j]], rg_v.at[b],
                                  gsem.at[b]).wait()

        def wait_scatter(j, b):
            pltpu.make_async_copy(rs_v.at[b], acc.at[dst_v.at[j]],
                                  ssem.at[b]).wait()

        for b in range(NBUF):
            start_gather(b, b)

        def wave(p, carry):
            for b in range(NBUF):
                j = NBUF * p + b
                wait_gather(j, b)

                @pl.when(j >= NBUF)
                def _():
                    wait_scatter(j - NBUF, b)

                zlane = lax.iota(jnp.int32, 16) * 0

                @plsc.parallel_loop(0, CHUNK // 16, unroll=2)
                def _scale_grp(g):
                    w16 = wts_v[j, pl.ds(g * 16, 16)]
                    for l in range(16):
                        wt = w16.at[zlane + l].get(mode="promise_in_bounds")
                        e = g * 16 + l
                        for k in range(HID // 16):
                            rs_v[b, e, pl.ds(k * 16, 16)] = \
                                rg_v[b, e, pl.ds(k * 16, 16)] * wt

                @pl.when(j + NBUF < cnt)
                def _():
                    start_gather(j + NBUF, b)

                pltpu.async_copy(rs_v.at[b], acc.at[dst_v.at[j]], ssem.at[b],
                                 add=True)
            return carry

        lax.fori_loop(0, cnt // NBUF, wave, 0)
        for b in range(NBUF):
            wait_scatter(cnt - NBUF + b, b)

    @pl.when(c == 0)
    def _():
        run_edges(s * CNT0, CNT0)

    @pl.when(c == 1)
    def _():
        run_edges(NS * CNT0 + s * CNT1, CNT1)

    plsc.subcore_barrier()

    @pl.when(c == 0)
    def _():
        def wout0(i, carry):
            pltpu.sync_copy(acc.at[pl.ds(base + i * CHUNK, CHUNK)],
                            out_hbm.at[pl.ds(base + i * CHUNK, CHUNK),
                                       pl.ds(0, HID)])
            return carry

        lax.fori_loop(0, ROWS_PER_W // CHUNK, wout0, 0)

    @pl.when(c == 1)
    def _():
        def wout1(i, carry):
            pltpu.sync_copy(acc.at[pl.ds(base + i * CHUNK, CHUNK)],
                            out_hbm.at[pl.ds(base + i * CHUNK, CHUNK),
                                       pl.ds(HID, HID)])
            return carry

        lax.fori_loop(0, ROWS_PER_W // CHUNK, wout1, 0)


@functools.lru_cache(maxsize=None)
def _spmm_call():
    return pl.kernel(
        _spmm_body,
        out_type=jax.ShapeDtypeStruct((ACCN, 2 * HID), jnp.float32),
        mesh=plsc.VectorSubcoreMesh(core_axis_name="c", subcore_axis_name="s",
                                    num_cores=NC, num_subcores=NS),
        scratch_types=[
            pltpu.VMEM((CNTMAX, CHUNK), jnp.int32),
            pltpu.VMEM((CNTMAX, CHUNK), jnp.int32),
            pltpu.VMEM((CNTMAX, CHUNK), jnp.float32),
            pltpu.VMEM((NBUF, CHUNK, HID), jnp.float32),
            pltpu.VMEM((NBUF, CHUNK, HID), jnp.float32),
            pltpu.VMEM_SHARED((ACCN, HID), jnp.float32),
            pltpu.SemaphoreType.DMA((NBUF,)),
            pltpu.SemaphoreType.DMA((NBUF,)),
        ],
        compiler_params=pltpu.CompilerParams(use_tc_tiling_on_sc=False),
    )


def _h0_kern(x_ref, w_ref, b_ref, o_ref):
    acc = jnp.dot(x_ref[...], w_ref[...], preferred_element_type=jnp.float32)
    o_ref[...] = jnp.maximum(acc + b_ref[...], 0.0)


def _layer_kern(p_ref, h0_ref, hl_ref, wc_ref, o_ref, *, theta, beta):
    p = p_ref[...]
    sup = (1.0 - ALPHA) * (p[:, :HID] + p[:, HID:]) + ALPHA * h0_ref[...]
    out = theta * jnp.dot(sup, wc_ref[...], preferred_element_type=jnp.float32) \
        + (1.0 - theta) * sup
    o_ref[...] = beta * jnp.maximum(out, 0.0) + (1.0 - beta) * hl_ref[...]


def _zn_kern(z_ref, o_ref):
    z = z_ref[...]
    n2 = jnp.sum(z * z, axis=1, keepdims=True)
    n = jnp.sqrt(n2)
    o_ref[...] = z / jnp.maximum(n, 1e-12)


def _sim_kern(zb_ref, znt_ref, loss_ref, acc_ref):
    i = pl.program_id(0)
    zb = zb_ref[...]
    s = jnp.dot(zb, znt_ref[...], preferred_element_type=jnp.float32)
    es = jnp.exp(s * (1.0 / TAU))
    rowsum = jnp.sum(es, axis=1, keepdims=True) - float(NPAD - N)
    d = jnp.sum(zb * zb, axis=1, keepdims=True)
    diag = jnp.exp(d * (1.0 / TAU))
    neg = rowsum - diag
    ct = -jnp.log(diag / (2.0 * neg))
    rid = i * SIMB + lax.broadcasted_iota(jnp.int32, (SIMB, 1), 0)
    blk = jnp.sum(jnp.where(rid < N, ct, 0.0))

    @pl.when(i == 0)
    def _():
        acc_ref[0] = 0.0

    acc_ref[0] += blk

    @pl.when(i == NPAD // SIMB - 1)
    def _():
        loss_ref[0, 0] = acc_ref[0] / float(N)


def _logits_kern(h_ref, w_ref, b_ref, o_ref):
    logits = jnp.dot(h_ref[...], w_ref[...],
                     preferred_element_type=jnp.float32) + b_ref[...]
    m = jnp.max(logits, axis=1, keepdims=True)
    sh = logits - m
    lse = jnp.log(jnp.sum(jnp.exp(sh), axis=1, keepdims=True))
    o_ref[...] = sh - lse


_h0_call = pl.pallas_call(
    _h0_kern,
    grid=(N // RB,),
    in_specs=[
        pl.BlockSpec((RB, NFEAT), lambda i: (i, 0)),
        pl.BlockSpec((NFEAT, HID), lambda i: (0, 0)),
        pl.BlockSpec((1, HID), lambda i: (0, 0)),
    ],
    out_specs=pl.BlockSpec((RB, HID), lambda i: (i, 0)),
    out_shape=jax.ShapeDtypeStruct((N, HID), jnp.float32),
)


def _layer_call(theta, beta):
    return pl.pallas_call(
        functools.partial(_layer_kern, theta=theta, beta=beta),
        grid=(N // RB,),
        in_specs=[
            pl.BlockSpec((RB, 2 * HID), lambda i: (i, 0)),
            pl.BlockSpec((RB, HID), lambda i: (i, 0)),
            pl.BlockSpec((RB, HID), lambda i: (i, 0)),
            pl.BlockSpec((HID, HID), lambda i: (0, 0)),
        ],
        out_specs=pl.BlockSpec((RB, HID), lambda i: (i, 0)),
        out_shape=jax.ShapeDtypeStruct((N, HID), jnp.float32),
    )


_zn_call = pl.pallas_call(
    _zn_kern,
    grid=(NPAD // 2048,),
    in_specs=[pl.BlockSpec((2048, HID), lambda i: (i, 0))],
    out_specs=pl.BlockSpec((2048, HID), lambda i: (i, 0)),
    out_shape=jax.ShapeDtypeStruct((NPAD, HID), jnp.float32),
)

_sim_call = pl.pallas_call(
    _sim_kern,
    grid=(NPAD // SIMB,),
    in_specs=[
        pl.BlockSpec((SIMB, HID), lambda i: (i, 0)),
        pl.BlockSpec((HID, NPAD), lambda i: (0, 0)),
    ],
    out_specs=pl.BlockSpec(memory_space=pltpu.SMEM),
    out_shape=jax.ShapeDtypeStruct((1, 1), jnp.float32),
    scratch_shapes=[pltpu.SMEM((1,), jnp.float32)],
)

_logits_call = pl.pallas_call(
    _logits_kern,
    grid=(N // RB,),
    in_specs=[
        pl.BlockSpec((RB, HID), lambda i: (i, 0)),
        pl.BlockSpec((HID, 128), lambda i: (0, 0)),
        pl.BlockSpec((1, 128), lambda i: (0, 0)),
    ],
    out_specs=pl.BlockSpec((RB, 128), lambda i: (i, 0)),
    out_shape=jax.ShapeDtypeStruct((N, 128), jnp.float32),
)


def kernel(x, edge_index, edge_weight, Wc, W0, b0, W1, b1):
    h0 = _h0_call(x, W0, b0.reshape(1, HID))

    dst = edge_index[0]
    src = edge_index[1]
    pad = EPAD - E
    src_p = jnp.concatenate([src, jnp.zeros((pad,), jnp.int32)]).reshape(TOTCH, CHUNK)
    dst_p = jnp.concatenate([dst, jnp.zeros((pad,), jnp.int32)]).reshape(TOTCH, CHUNK)
    wts_p = jnp.concatenate([edge_weight, jnp.zeros((pad,), jnp.float32)]).reshape(TOTCH, CHUNK)

    last = h0
    for i in range(NLAYER):
        l = i + 1
        theta = math.log(LAM / l + 1.0)
        beta = math.log(LAMDA / l + 1.0)
        parts = _spmm_call()(last, src_p, dst_p, wts_p)
        last = _layer_call(theta, beta)(parts, h0, last, Wc[i])

    lastp = jnp.pad(last, ((0, NPAD - N), (0, 0)))
    znp = _zn_call(lastp)
    loss = _sim_call(znp, znp.T)[0, 0]

    W1p = jnp.pad(W1, ((0, 0), (0, 128 - NCLASS)))
    b1p = jnp.pad(b1, (0, 128 - NCLASS), constant_values=-1e30).reshape(1, 128)
    logp = _logits_call(last, W1p, b1p)[:, :NCLASS]
    return (logp, loss)
